# baseline (speedup 1.0000x reference)
import functools

import jax
import jax.numpy as jnp
from jax import lax
from jax.experimental import pallas as pl
from jax.experimental.pallas import tpu as pltpu
from jax.experimental.pallas import tpu_sc as plsc

N = 10000
E = 320000
D = 128
DH = D // 2
NEG_SLOPE = 0.2

NC = 2
NS = 16
NW = NC * NS
C = 128
NCHUNK = E // C
NKMAX = (NCHUNK + NW - 1) // NW
ROWS_PER_TILE = 624
TAIL_ROWS = N - NS * ROWS_PER_TILE


def _project_body(x_ref, w_ref, a2_ref, xlin_ref, aa_ref):
    xl = jnp.dot(x_ref[...], w_ref[...], preferred_element_type=jnp.float32)
    xlin_ref[...] = xl
    aa_ref[...] = jnp.dot(xl, a2_ref[...], preferred_element_type=jnp.float32)


def _project(x, W, a2):
    return pl.pallas_call(
        _project_body,
        out_shape=(
            jax.ShapeDtypeStruct((N, D), jnp.float32),
            jax.ShapeDtypeStruct((N, D), jnp.float32),
        ),
    )(x, W, a2)


def _sc_body(a_src_hbm, a_dst_hbm, edge_hbm, xlin2_hbm,
             msg_hbm, den_hbm,
             a_src_v, a_dst_v, den_v,
             src_v0, src_v1, dst_v0, dst_v1, idx2_v0, idx2_v1,
             rows_v0, rows_v1, wcache_v, msg_v0, msg_v1, acc_sh,
             semi0, semi1, semg0, semg1, sems0, sems1):
    cid = lax.axis_index("c")
    sid = lax.axis_index("s")
    wid = sid * NC + cid

    srcs = (src_v0, src_v1)
    dsts = (dst_v0, dst_v1)
    idx2s = (idx2_v0, idx2_v1)
    rows = (rows_v0, rows_v1)
    msgs = (msg_v0, msg_v1)
    semI = (semi0, semi1)
    semG = (semg0, semg1)
    semS = (sems0, sems1)

    pltpu.sync_copy(a_src_hbm, a_src_v)
    pltpu.sync_copy(a_dst_hbm, a_dst_v)

    zeros16 = jnp.zeros((16,), jnp.float32)

    def zero_den(r, _):
        den_v[pl.ds(r * 16, 16)] = zeros16
        return 0

    lax.fori_loop(0, N // 16, zero_den, 0)

    base_row = sid * ROWS_PER_TILE
    nk = (NCHUNK - wid + NW - 1) // NW

    def start_idx(i, b):
        base = (wid + i * NW) * C
        pltpu.async_copy(edge_hbm.at[0, pl.ds(base, C)], srcs[b], semI[b])
        pltpu.async_copy(edge_hbm.at[1, pl.ds(base, C)], dsts[b], semI[b])

    def phase(p, _):
        def zero_row(r, _):
            for c16 in range(DH // 16):
                msg_v0[r, pl.ds(c16 * 16, 16)] = zeros16
            return 0

        lax.fori_loop(0, C, zero_row, 0)

        for i in range(4):
            pltpu.sync_copy(msg_v0, acc_sh.at[pl.ds(base_row + i * C, C)])
        rem = ROWS_PER_TILE - 4 * C
        pltpu.sync_copy(msg_v0.at[pl.ds(0, rem)],
                        acc_sh.at[pl.ds(base_row + 4 * C, rem)])

        @pl.when(sid == NS - 1)
        def _zero_tail():
            pltpu.sync_copy(msg_v0.at[pl.ds(0, TAIL_ROWS)],
                            acc_sh.at[pl.ds(NS * ROWS_PER_TILE, TAIL_ROWS)])

        plsc.subcore_barrier()

        start_idx(0, 0)

        def pair(ii, _):
            for b in range(2):
                o = 1 - b
                i = 2 * ii + b

                @pl.when(i < nk)
                def _chunk():
                    pltpu.make_async_copy(edge_hbm.at[0, pl.ds(0, C)],
                                          srcs[b], semI[b]).wait()
                    pltpu.make_async_copy(edge_hbm.at[1, pl.ds(0, C)],
                                          dsts[b], semI[b]).wait()

                    def mini(g, _):
                        sl = pl.ds(g * 16, 16)
                        idx2s[b][sl] = srcs[b][sl] * 2 + p
                        return 0

                    lax.fori_loop(0, C // 16, mini, 0)
                    gcp = pltpu.async_copy(xlin2_hbm.at[idx2s[b]],
                                           rows[b], semG[b])

                    @pl.when(i >= 1)
                    def _drain():
                        pltpu.make_async_copy(
                            msgs[o], acc_sh.at[dsts[o]], semS[o]).wait()

                    @pl.when(i + 1 < nk)
                    def _prefetch():
                        start_idx(i + 1, o)

                    @pl.when(p == 0)
                    def _weights():
                        def grp(g, _):
                            sl = pl.ds(g * 16, 16)
                            sidx = srcs[b][sl]
                            didx = dsts[b][sl]
                            logit = (plsc.load_gather(a_src_v, [sidx])
                                     + plsc.load_gather(a_dst_v, [didx]))
                            w16 = jnp.exp(jnp.where(logit >= 0, logit,
                                                    NEG_SLOPE * logit))
                            wcache_v[pl.ds(i * C + g * 16, 16)] = w16
                            plsc.addupdate_scatter(den_v, [didx], w16)
                            return 0

                        lax.fori_loop(0, C // 16, grp, 0)

                    gcp.wait()

                    @plsc.parallel_loop(0, C // 16, unroll=2)
                    def edge16(g):
                        w16 = wcache_v[pl.ds(i * C + g * 16, 16)]
                        for j in range(16):
                            e = g * 16 + j
                            wv = jnp.full((16,), w16[j], jnp.float32)
                            for c16 in range(DH // 16):
                                sl = pl.ds(c16 * 16, 16)
                                msgs[b][e, sl] = rows[b][e, sl] * wv

                    pltpu.async_copy(msgs[b], acc_sh.at[dsts[b]],
                                     semS[b], add=True)

            return 0

        lax.fori_loop(0, (nk + 1) // 2, pair, 0)

        @pl.when(nk % 2 == 1)
        def _d0():
            pltpu.make_async_copy(msgs[0], acc_sh.at[dsts[0]],
                                  semS[0]).wait()

        @pl.when(nk % 2 == 0)
        def _d1():
            pltpu.make_async_copy(msgs[1], acc_sh.at[dsts[1]],
                                  semS[1]).wait()

        plsc.subcore_barrier()

        pltpu.sync_copy(acc_sh.at[pl.ds(base_row, ROWS_PER_TILE)],
                        msg_hbm.at[cid, p, pl.ds(base_row, ROWS_PER_TILE)])

        @pl.when(sid == NS - 1)
        def _flush_tail():
            pltpu.sync_copy(
                acc_sh.at[pl.ds(NS * ROWS_PER_TILE, TAIL_ROWS)],
                msg_hbm.at[cid, p, pl.ds(NS * ROWS_PER_TILE, TAIL_ROWS)])

        plsc.subcore_barrier()
        return 0

    lax.fori_loop(0, 2, phase, 0)

    pltpu.sync_copy(den_v, den_hbm.at[wid])


_sc_edges = functools.partial(
    pl.kernel,
    out_type=(
        jax.ShapeDtypeStruct((NC, 2, N, DH), jnp.float32),
        jax.ShapeDtypeStruct((NW, N), jnp.float32),
    ),
    mesh=plsc.VectorSubcoreMesh(core_axis_name="c", subcore_axis_name="s"),
    compiler_params=pltpu.CompilerParams(
        needs_layout_passes=False, use_tc_tiling_on_sc=False),
    scratch_types=[
        pltpu.VMEM((N,), jnp.float32),
        pltpu.VMEM((N,), jnp.float32),
        pltpu.VMEM((N,), jnp.float32),
        pltpu.VMEM((C,), jnp.int32),
        pltpu.VMEM((C,), jnp.int32),
        pltpu.VMEM((C,), jnp.int32),
        pltpu.VMEM((C,), jnp.int32),
        pltpu.VMEM((C,), jnp.int32),
        pltpu.VMEM((C,), jnp.int32),
        pltpu.VMEM((C, DH), jnp.float32),
        pltpu.VMEM((C, DH), jnp.float32),
        pltpu.VMEM((NKMAX * C,), jnp.float32),
        pltpu.VMEM((C, DH), jnp.float32),
        pltpu.VMEM((C, DH), jnp.float32),
        pltpu.VMEM_SHARED((N, DH), jnp.float32),
        pltpu.SemaphoreType.DMA,
        pltpu.SemaphoreType.DMA,
        pltpu.SemaphoreType.DMA,
        pltpu.SemaphoreType.DMA,
        pltpu.SemaphoreType.DMA,
        pltpu.SemaphoreType.DMA,
    ],
)(_sc_body)


def _combine_body(msg_ref, den_ref, xlin_ref, aa_ref, out_ref):
    den = jnp.sum(den_ref[...], axis=0)[:, None]
    asum = aa_ref[:, 0:1] + aa_ref[:, 1:2]
    wself = jnp.exp(jnp.where(asum >= 0, asum, NEG_SLOPE * asum))
    inv = 1.0 / (den + wself + 1e-16)
    msgl = msg_ref[0, 0] + msg_ref[1, 0]
    msgr = msg_ref[0, 1] + msg_ref[1, 1]
    out_ref[:, :DH] = (msgl + wself * xlin_ref[:, :DH]) * inv
    out_ref[:, DH:] = (msgr + wself * xlin_ref[:, DH:]) * inv


def _combine(msg, den, xlin, aa):
    return pl.pallas_call(
        _combine_body,
        out_shape=jax.ShapeDtypeStruct((N, D), jnp.float32),
    )(msg, den, xlin, aa)


def kernel(x, edge_index, W, att_src, att_dst):
    a2 = jnp.zeros((D, D), jnp.float32)
    a2 = a2.at[:, 0].set(att_src.reshape(-1)).at[:, 1].set(att_dst.reshape(-1))

    xlin, aa = _project(x, W, a2)
    a_src = aa[:, 0]
    a_dst = aa[:, 1]
    xlin2 = xlin.reshape(2 * N, DH)

    msg, den = _sc_edges(a_src, a_dst, edge_index, xlin2)
    return _combine(msg, den, xlin, aa)
